# baseline (speedup 1.0000x reference)
import functools

import jax
import jax.numpy as jnp
from jax import lax
from jax.experimental import pallas as pl
from jax.experimental.pallas import tpu as pltpu
from jax.experimental.pallas import tpu_sc as plsc

N_NODES = 10000
N_EDGES = 320000
D = 128

NUM_CORES = 2
NUM_SUBCORES = 16
NUM_WORKERS = NUM_CORES * NUM_SUBCORES
BLK = 80
NBLK = 128
EDGES_PAD = NUM_WORKERS * NBLK * BLK
ACC_ROWS = 10240
ROWS_PER_TILE = ACC_ROWS // NUM_SUBCORES


CH = 32
NCH = NBLK // CH


def _sc_agg_body(h_hbm, src_hbm, dst_hbm, zeros_hbm, out_hbm,
                 sidxA, didxA, sidxB, didxB, rows0, rows1, acc, s0, s1, si):
    cid = lax.axis_index("c")
    sid = lax.axis_index("s")
    wid = cid * NUM_SUBCORES + sid

    def idx_load(c, sbuf, dbuf):
        pltpu.async_copy(src_hbm.at[wid, pl.ds(c * CH, CH)], sbuf, si)
        pltpu.async_copy(dst_hbm.at[wid, pl.ds(c * CH, CH)], dbuf, si)

    def idx_wait(c, sbuf, dbuf):
        pltpu.make_async_copy(src_hbm.at[wid, pl.ds(c * CH, CH)], sbuf, si).wait()
        pltpu.make_async_copy(dst_hbm.at[wid, pl.ds(c * CH, CH)], dbuf, si).wait()

    idx_load(0, sidxA, didxA)
    pltpu.sync_copy(zeros_hbm, acc.at[pl.ds(sid * ROWS_PER_TILE, ROWS_PER_TILE)])
    idx_wait(0, sidxA, didxA)
    plsc.subcore_barrier()

    def gather(sbuf, j, buf, sem):
        pltpu.async_copy(h_hbm.at[sbuf.at[j]], buf, sem)

    def gather_wait(sbuf, j, buf, sem):
        pltpu.make_async_copy(h_hbm.at[sbuf.at[j]], buf, sem).wait()

    def scatter(dbuf, j, buf):
        pltpu.sync_copy(buf, acc.at[dbuf.at[j]], add=True)

    for c in range(NCH):
        sbuf, dbuf = (sidxA, didxA) if c % 2 == 0 else (sidxB, didxB)
        nbuf, mbuf = (sidxB, didxB) if c % 2 == 0 else (sidxA, didxA)
        if c + 1 < NCH:
            idx_load(c + 1, nbuf, mbuf)

        gather(sbuf, 0, rows0, s0)
        gather(sbuf, 1, rows1, s1)

        @pl.loop(0, CH - 2, step=2)
        def _(j):
            gather_wait(sbuf, j, rows0, s0)
            scatter(dbuf, j, rows0)
            gather(sbuf, j + 2, rows0, s0)
            gather_wait(sbuf, j + 1, rows1, s1)
            scatter(dbuf, j + 1, rows1)
            gather(sbuf, j + 3, rows1, s1)

        gather_wait(sbuf, CH - 2, rows0, s0)
        scatter(dbuf, CH - 2, rows0)
        gather_wait(sbuf, CH - 1, rows1, s1)
        scatter(dbuf, CH - 1, rows1)

        if c + 1 < NCH:
            idx_wait(c + 1, nbuf, mbuf)

    plsc.subcore_barrier()
    row0 = sid * ROWS_PER_TILE
    pltpu.sync_copy(acc.at[pl.ds(row0, ROWS_PER_TILE)],
                    out_hbm.at[cid, pl.ds(row0, ROWS_PER_TILE)])


@jax.jit
def _sc_agg(h, src, dst, zeros):
    mesh = plsc.VectorSubcoreMesh(core_axis_name="c", subcore_axis_name="s")
    k = pl.kernel(
        _sc_agg_body,
        out_type=jax.ShapeDtypeStruct((NUM_CORES, ACC_ROWS, D), jnp.float32),
        mesh=mesh,
        scratch_types=[
            pltpu.VMEM((CH, BLK), jnp.int32),
            pltpu.VMEM((CH, BLK), jnp.int32),
            pltpu.VMEM((CH, BLK), jnp.int32),
            pltpu.VMEM((CH, BLK), jnp.int32),
            pltpu.VMEM((BLK, D), jnp.float32),
            pltpu.VMEM((BLK, D), jnp.float32),
            pltpu.VMEM_SHARED((ACC_ROWS, D), jnp.float32),
            pltpu.SemaphoreType.DMA,
            pltpu.SemaphoreType.DMA,
            pltpu.SemaphoreType.DMA,
        ],
    )
    return k(h, src.reshape(NUM_WORKERS, NBLK, BLK),
             dst.reshape(NUM_WORKERS, NBLK, BLK), zeros)


ROW_BLK = 1000


def _tc_mlp_body(x_ref, p_ref, wt_ref, b_ref, o_ref, *, relu):
    s = x_ref[...] + p_ref[0] + p_ref[1]
    y = jnp.dot(s, wt_ref[...], preferred_element_type=jnp.float32) + b_ref[...]
    if relu:
        y = jnp.maximum(y, 0.0)
    o_ref[...] = y


def _tc_mlp(x, parts, wt, b, relu):
    grid = (N_NODES // ROW_BLK,)
    return pl.pallas_call(
        functools.partial(_tc_mlp_body, relu=relu),
        grid=grid,
        in_specs=[
            pl.BlockSpec((ROW_BLK, D), lambda i: (i, 0)),
            pl.BlockSpec((NUM_CORES, ROW_BLK, D), lambda i: (0, i, 0)),
            pl.BlockSpec((D, D), lambda i: (0, 0)),
            pl.BlockSpec((1, D), lambda i: (0, 0)),
        ],
        out_specs=pl.BlockSpec((ROW_BLK, D), lambda i: (i, 0)),
        out_shape=jax.ShapeDtypeStruct((N_NODES, D), jnp.float32),
    )(x, parts, wt, b)


def kernel(x, edge_index, W1, b1, W2, b2):
    npad = EDGES_PAD - N_EDGES
    src = jnp.concatenate(
        [edge_index[0].astype(jnp.int32), jnp.zeros((npad,), jnp.int32)])
    dst = jnp.concatenate(
        [edge_index[1].astype(jnp.int32),
         jnp.full((npad,), ACC_ROWS - 1, jnp.int32)])
    zeros = jnp.zeros((ROWS_PER_TILE, D), jnp.float32)

    agg1 = _sc_agg(x, src, dst, zeros)
    h = _tc_mlp(x, agg1, W1.T, b1.reshape(1, D), relu=True)
    agg2 = _sc_agg(h, src, dst, zeros)
    out = _tc_mlp(h, agg2, W2.T, b2.reshape(1, D), relu=False)
    return out
